# baseline (speedup 1.0000x reference)
import functools

import jax
import jax.numpy as jnp
from jax import lax
from jax.experimental import pallas as pl
from jax.experimental.pallas import tpu as pltpu
from jax.experimental.pallas import tpu_sc as plsc

_NUM_NODES = 1000000
_DIM = 64
_BATCH = 16384

_NC = 2
_NS = 16
_NW = _NC * _NS
_BPW = _BATCH // _NW
_CHUNK = 128
_NCH = _BPW // _CHUNK
_PADDED = 2 * _DIM

_mesh = plsc.VectorSubcoreMesh(core_axis_name="c", subcore_axis_name="s")


@functools.partial(
    pl.kernel,
    mesh=_mesh,
    out_type=(
        jax.ShapeDtypeStruct((_BATCH, _PADDED), jnp.float32),
        jax.ShapeDtypeStruct((_BATCH,), jnp.float32),
    ),
    scratch_types=[
        pltpu.VMEM((_NCH, _CHUNK), jnp.int32),
        pltpu.VMEM((_NCH, _CHUNK, _PADDED), jnp.float32),
        pltpu.VMEM((_NCH, _CHUNK), jnp.float32),
        pltpu.SemaphoreType.DMA,
        pltpu.SemaphoreType.DMA,
    ],
)
def _tgn_gather(n_id_hbm, memp_hbm, lu_hbm, mem_out, lu_out,
                idx_v, rows_v, lu_v, sem_m, sem_l):
    wid = lax.axis_index("s") * _NC + lax.axis_index("c")
    base = wid * _BPW
    pltpu.sync_copy(n_id_hbm.at[pl.ds(wid * _NCH, _NCH)], idx_v)
    lu_copies = [
        pltpu.async_copy(lu_hbm.at[idx_v.at[j]], lu_v.at[j], sem_l)
        for j in range(_NCH)
    ]
    row_copies = [
        pltpu.async_copy(memp_hbm.at[idx_v.at[j]], rows_v.at[j], sem_m)
        for j in range(_NCH)
    ]
    for j in range(_NCH):
        row_copies[j].wait()
        pltpu.sync_copy(
            rows_v.at[j],
            mem_out.at[pl.ds(base + j * _CHUNK, _CHUNK)],
        )
    for j in range(_NCH):
        lu_copies[j].wait()
        pltpu.sync_copy(lu_v.at[j], lu_out.at[pl.ds(base + j * _CHUNK, _CHUNK)])


def kernel(n_id, memory, last_update):
    n_id2 = n_id.astype(jnp.int32).reshape(_NW * _NCH, _CHUNK)
    proj = jnp.concatenate([jnp.eye(_DIM, dtype=jnp.float32), jnp.zeros((_DIM, _DIM), jnp.float32)], axis=1)
    memp = jax.lax.dot(memory, proj, precision=jax.lax.Precision.HIGHEST)
    mem_out, lu_out = _tgn_gather(n_id2, memp, last_update)
    return (mem_out[:, :_DIM], lu_out)


# speedup vs baseline: 3.1447x; 3.1447x over previous
import functools

import jax
import jax.numpy as jnp
from jax import lax
from jax.experimental import pallas as pl
from jax.experimental.pallas import tpu as pltpu
from jax.experimental.pallas import tpu_sc as plsc

_NUM_NODES = 1000000
_DIM = 64
_BATCH = 16384

_NC = 2
_NS = 16
_NW = _NC * _NS
_BPW = _BATCH // _NW
_CHUNK = 128
_NCH = _BPW // _CHUNK
_PADDED = 2 * _DIM

_mesh = plsc.VectorSubcoreMesh(core_axis_name="c", subcore_axis_name="s")


@functools.partial(
    pl.kernel,
    mesh=_mesh,
    out_type=(
        jax.ShapeDtypeStruct((_BATCH, _PADDED), jnp.float32),
        jax.ShapeDtypeStruct((_BATCH,), jnp.float32),
    ),
    scratch_types=[
        pltpu.VMEM((_NCH, _CHUNK), jnp.int32),
        pltpu.VMEM((_NCH, _CHUNK, _PADDED), jnp.float32),
        pltpu.VMEM((_NCH, _CHUNK), jnp.float32),
        pltpu.SemaphoreType.DMA,
        pltpu.SemaphoreType.DMA,
    ],
)
def _tgn_gather(n_id_hbm, memp_hbm, lu_hbm, mem_out, lu_out,
                idx_v, rows_v, lu_v, sem_m, sem_l):
    wid = lax.axis_index("s") * _NC + lax.axis_index("c")
    base = wid * _BPW
    pltpu.sync_copy(n_id_hbm.at[pl.ds(wid * _NCH, _NCH)], idx_v)
    lu_copies = [
        pltpu.async_copy(lu_hbm.at[idx_v.at[j]], lu_v.at[j], sem_l)
        for j in range(_NCH)
    ]
    row_copies = [
        pltpu.async_copy(memp_hbm.at[idx_v.at[j]], rows_v.at[j], sem_m)
        for j in range(_NCH)
    ]
    for j in range(_NCH):
        row_copies[j].wait()
        pltpu.sync_copy(
            rows_v.at[j],
            mem_out.at[pl.ds(base + j * _CHUNK, _CHUNK)],
        )
    for j in range(_NCH):
        lu_copies[j].wait()
        pltpu.sync_copy(lu_v.at[j], lu_out.at[pl.ds(base + j * _CHUNK, _CHUNK)])


def kernel(n_id, memory, last_update):
    n_id2 = n_id.astype(jnp.int32).reshape(_NW * _NCH, _CHUNK)
    proj = jnp.concatenate([jnp.eye(_DIM, dtype=jnp.float32), jnp.zeros((_DIM, _DIM), jnp.float32)], axis=1)
    memp = jax.lax.dot(memory, proj)
    mem_out, lu_out = _tgn_gather(n_id2, memp, last_update)
    return (mem_out[:, :_DIM], lu_out)
